# Initial kernel scaffold; baseline (speedup 1.0000x reference)
import jax
import jax.numpy as jnp
from jax.experimental import pallas as pl


def kernel(x, edge_index, batch, Wm1, bm1, Wm2, bm2, Wf, bf, Wo1, bo1, Wo2, bo2):
    raise NotImplementedError("write your pallas kernel here")


# speedup vs baseline: 5.6191x; 5.6191x over previous
import functools

import jax
import jax.numpy as jnp
from jax import lax
from jax.experimental import pallas as pl
from jax.experimental.pallas import tpu as pltpu
from jax.experimental.pallas import tpu_sc as plsc

N = 10000
E = 320000
D = 128
OUT = 128

NP = 10240
DR = NP // D
NW = 32
EPW = E // NW
K = 80
CHUNKS = EPW // K
RPT = NP // 16

S1_BLK = 1024
S3_BLK = 1024


def _s1_body(x_ref, wm1_ref, bm1_ref, wm2_ref, bm2_ref, o_ref):
    h = jnp.maximum(x_ref[...] @ wm1_ref[...] + bm1_ref[...], 0.0)
    o_ref[...] = h @ wm2_ref[...] + bm2_ref[...]


def _stage1(x_pad, Wm1, bm1, Wm2, bm2):
    grid = NP // S1_BLK
    return pl.pallas_call(
        _s1_body,
        grid=(grid,),
        in_specs=[
            pl.BlockSpec((S1_BLK, D), lambda i: (i, 0)),
            pl.BlockSpec((D, D), lambda i: (0, 0)),
            pl.BlockSpec((1, D), lambda i: (0, 0)),
            pl.BlockSpec((D, D), lambda i: (0, 0)),
            pl.BlockSpec((1, D), lambda i: (0, 0)),
        ],
        out_specs=pl.BlockSpec((S1_BLK, D), lambda i: (i, 0)),
        out_shape=jax.ShapeDtypeStruct((NP, D), jnp.float32),
    )(x_pad, Wm1, bm1.reshape(1, D), Wm2, bm2.reshape(1, D))


def _sc_scatter_body(msg_hbm, row_hbm, col_hbm, zeros_hbm,
                     agg_out, deg_out,
                     ridx_v, cidx_v, rows_v, deg_v, tmp_v, acc_v,
                     agg_sh, deg_all_sh, sem):
    c = lax.axis_index("c")
    s = lax.axis_index("s")
    wid = s * 2 + c

    pltpu.sync_copy(zeros_hbm.at[pl.ds(s * RPT, RPT)],
                    agg_sh.at[pl.ds(s * RPT, RPT)])

    def zinit(i, carry):
        deg_v[pl.ds(i * 16, 16)] = jnp.zeros((16,), jnp.float32)
        return carry

    lax.fori_loop(0, NP // 16, zinit, 0)

    plsc.subcore_barrier()

    ones = jnp.full((16,), 1.0, jnp.float32)

    def body(i, carry):
        off = wid * EPW + i * K
        pltpu.sync_copy(row_hbm.at[pl.ds(off, K)], ridx_v)
        pltpu.sync_copy(col_hbm.at[pl.ds(off, K)], cidx_v)
        pltpu.async_copy(msg_hbm.at[ridx_v], rows_v, sem).wait()
        pltpu.sync_copy(rows_v, agg_sh.at[cidx_v], add=True)
        for j in range(K // 16):
            cc = cidx_v[pl.ds(j * 16, 16)]
            plsc.addupdate_scatter(deg_v, [cc], ones)
        return carry

    lax.fori_loop(0, CHUNKS, body, 0)

    pltpu.sync_copy(deg_v, deg_all_sh.at[s])
    plsc.subcore_barrier()

    pltpu.sync_copy(deg_all_sh.at[0, pl.ds(s * RPT, RPT)], acc_v)

    def red(t, carry):
        pltpu.sync_copy(deg_all_sh.at[t, pl.ds(s * RPT, RPT)], tmp_v)
        for k in range(RPT // 16):
            sl = pl.ds(k * 16, 16)
            acc_v[sl] = acc_v[sl] + tmp_v[sl]
        return carry

    lax.fori_loop(1, 16, red, 0)

    pltpu.sync_copy(agg_sh.at[pl.ds(s * RPT, RPT)],
                    agg_out.at[c, pl.ds(s * RPT, RPT)])
    pltpu.sync_copy(acc_v, deg_out.at[c, pl.ds(s * RPT, RPT)])


def _stage2(msg, row, col, zeros):
    kern = functools.partial(
        pl.kernel,
        mesh=plsc.VectorSubcoreMesh(core_axis_name="c", subcore_axis_name="s"),
        out_type=(
            jax.ShapeDtypeStruct((2, NP, D), jnp.float32),
            jax.ShapeDtypeStruct((2, NP), jnp.float32),
        ),
        scratch_types=[
            pltpu.VMEM((K,), jnp.int32),
            pltpu.VMEM((K,), jnp.int32),
            pltpu.VMEM((K, D), jnp.float32),
            pltpu.VMEM((NP,), jnp.float32),
            pltpu.VMEM((RPT,), jnp.float32),
            pltpu.VMEM((RPT,), jnp.float32),
            pltpu.VMEM_SHARED((NP, D), jnp.float32),
            pltpu.VMEM_SHARED((16, NP), jnp.float32),
            pltpu.SemaphoreType.DMA,
        ],
        compiler_params=pltpu.CompilerParams(needs_layout_passes=False),
    )(_sc_scatter_body)
    return kern(msg, row, col, zeros)


def _s3_body(x_ref, a0_ref, a1_ref, deg_ref, wf_ref, bf_ref, wo1a_ref,
             wo1b_ref, bo1_ref, wo2_ref, bo2_ref, o_ref):
    s = a0_ref[...] + a1_ref[...]
    aggn = s / jnp.maximum(deg_ref[...], 1.0)
    fx = x_ref[...] + jnp.tanh(aggn @ wf_ref[...] + bf_ref[...])
    o = jnp.maximum(fx @ wo1a_ref[...] + aggn @ wo1b_ref[...] + bo1_ref[...], 0.0)
    o_ref[...] = o @ wo2_ref[...] + bo2_ref[...]


def _stage3(x_pad, a0, a1, degb, Wf, bf, Wo1, bo1, Wo2, bo2):
    grid = NP // S3_BLK
    return pl.pallas_call(
        _s3_body,
        grid=(grid,),
        in_specs=[
            pl.BlockSpec((S3_BLK, D), lambda i: (i, 0)),
            pl.BlockSpec((S3_BLK, D), lambda i: (i, 0)),
            pl.BlockSpec((S3_BLK, D), lambda i: (i, 0)),
            pl.BlockSpec((S3_BLK, D), lambda i: (i, 0)),
            pl.BlockSpec((D, D), lambda i: (0, 0)),
            pl.BlockSpec((1, D), lambda i: (0, 0)),
            pl.BlockSpec((D, D), lambda i: (0, 0)),
            pl.BlockSpec((D, D), lambda i: (0, 0)),
            pl.BlockSpec((1, D), lambda i: (0, 0)),
            pl.BlockSpec((D, OUT), lambda i: (0, 0)),
            pl.BlockSpec((1, OUT), lambda i: (0, 0)),
        ],
        out_specs=pl.BlockSpec((S3_BLK, OUT), lambda i: (i, 0)),
        out_shape=jax.ShapeDtypeStruct((NP, OUT), jnp.float32),
    )(x_pad, a0, a1, degb, Wf, bf.reshape(1, D), Wo1[:D], Wo1[D:],
      bo1.reshape(1, D), Wo2, bo2.reshape(1, OUT))


def kernel(x, edge_index, batch, Wm1, bm1, Wm2, bm2, Wf, bf, Wo1, bo1, Wo2, bo2):
    x_pad = jnp.zeros((NP, D), jnp.float32).at[:N].set(x)
    msg = _stage1(x_pad, Wm1, bm1, Wm2, bm2)
    row = edge_index[0]
    col = edge_index[1]
    zeros = jnp.zeros((NP, D), jnp.float32)
    agg, deg = _stage2(msg, row, col, zeros)
    degb = jnp.broadcast_to((deg[0] + deg[1]).reshape(NP, 1), (NP, D))
    out = _stage3(x_pad, agg[0], agg[1], degb, Wf, bf, Wo1, bo1, Wo2, bo2)
    return out[:N]


# speedup vs baseline: 8.9519x; 1.5931x over previous
import functools

import jax
import jax.numpy as jnp
from jax import lax
from jax.experimental import pallas as pl
from jax.experimental.pallas import tpu as pltpu
from jax.experimental.pallas import tpu_sc as plsc

N = 10000
E = 320000
D = 128
OUT = 128

NP = 10240
DR = NP // D
NW = 32
EPW = E // NW
K = 80
CHUNKS = EPW // K
RPT = NP // 16

S1_BLK = 1024
S3_BLK = 1024


def _s1_body(x_ref, wm1_ref, bm1_ref, wm2_ref, bm2_ref, o_ref):
    h = jnp.maximum(x_ref[...] @ wm1_ref[...] + bm1_ref[...], 0.0)
    o_ref[...] = h @ wm2_ref[...] + bm2_ref[...]


def _stage1(x_pad, Wm1, bm1, Wm2, bm2):
    grid = NP // S1_BLK
    return pl.pallas_call(
        _s1_body,
        grid=(grid,),
        in_specs=[
            pl.BlockSpec((S1_BLK, D), lambda i: (i, 0)),
            pl.BlockSpec((D, D), lambda i: (0, 0)),
            pl.BlockSpec((1, D), lambda i: (0, 0)),
            pl.BlockSpec((D, D), lambda i: (0, 0)),
            pl.BlockSpec((1, D), lambda i: (0, 0)),
        ],
        out_specs=pl.BlockSpec((S1_BLK, D), lambda i: (i, 0)),
        out_shape=jax.ShapeDtypeStruct((NP, D), jnp.float32),
    )(x_pad, Wm1, bm1.reshape(1, D), Wm2, bm2.reshape(1, D))


def _sc_scatter_body(msg_hbm, row_hbm, col_hbm, zeros_hbm,
                     agg_out, deg_out, degall_out,
                     ridx_all, cidx_a, cidx_b, rows_a, rows_b,
                     deg_v, tmp_v, acc_v, agg_sh,
                     semg_a, semg_b, sems_a, sems_b, semi_a, semi_b):
    c = lax.axis_index("c")
    s = lax.axis_index("s")
    wid = s * 2 + c

    pltpu.sync_copy(zeros_hbm.at[pl.ds(s * RPT, RPT)],
                    agg_sh.at[pl.ds(s * RPT, RPT)])

    pltpu.sync_copy(row_hbm.at[pl.ds(wid * EPW, EPW)], ridx_all)

    def zinit(i, carry):
        deg_v[pl.ds(i * 16, 16)] = jnp.zeros((16,), jnp.float32)
        return carry

    lax.fori_loop(0, NP // 16, zinit, 0)

    plsc.subcore_barrier()

    ones = jnp.full((16,), 1.0, jnp.float32)

    def gstart(i, rows, semg):
        pltpu.async_copy(msg_hbm.at[ridx_all.at[pl.ds(i * K, K)]], rows, semg)

    def gwait(rows, semg):
        pltpu.make_async_copy(msg_hbm.at[ridx_all.at[pl.ds(0, K)]], rows,
                              semg).wait()

    def istart(i, cbuf, semi):
        pltpu.async_copy(col_hbm.at[pl.ds(wid * EPW + i * K, K)], cbuf, semi)

    def iwait(cbuf, semi):
        pltpu.make_async_copy(col_hbm.at[pl.ds(0, K)], cbuf, semi).wait()

    def sstart(rows, cbuf, sems):
        pltpu.async_copy(rows, agg_sh.at[cbuf], sems, add=True)

    def swait(rows, cbuf, sems):
        pltpu.make_async_copy(rows, agg_sh.at[cbuf], sems).wait()

    def dodeg(cbuf):
        for j in range(K // 16):
            cc = cbuf[pl.ds(j * 16, 16)]
            plsc.addupdate_scatter(deg_v, [cc], ones)

    pltpu.sync_copy(col_hbm.at[pl.ds(wid * EPW, K)], cidx_a)
    gstart(0, rows_a, semg_a)
    istart(1, cidx_b, semi_b)
    gwait(rows_a, semg_a)
    dodeg(cidx_a)
    sstart(rows_a, cidx_a, sems_a)
    iwait(cidx_b, semi_b)
    gstart(1, rows_b, semg_b)

    def pair(j, carry):
        i1 = 2 * j + 1
        swait(rows_a, cidx_a, sems_a)
        istart(i1 + 1, cidx_a, semi_a)
        gwait(rows_b, semg_b)
        dodeg(cidx_b)
        sstart(rows_b, cidx_b, sems_b)
        iwait(cidx_a, semi_a)
        gstart(i1 + 1, rows_a, semg_a)
        i2 = 2 * j + 2
        swait(rows_b, cidx_b, sems_b)
        istart(i2 + 1, cidx_b, semi_b)
        gwait(rows_a, semg_a)
        dodeg(cidx_a)
        sstart(rows_a, cidx_a, sems_a)
        iwait(cidx_b, semi_b)
        gstart(i2 + 1, rows_b, semg_b)
        return carry

    lax.fori_loop(0, (CHUNKS - 3) // 2, pair, 0)

    swait(rows_a, cidx_a, sems_a)
    istart(CHUNKS - 1, cidx_a, semi_a)
    gwait(rows_b, semg_b)
    dodeg(cidx_b)
    sstart(rows_b, cidx_b, sems_b)
    iwait(cidx_a, semi_a)
    gstart(CHUNKS - 1, rows_a, semg_a)
    swait(rows_b, cidx_b, sems_b)
    gwait(rows_a, semg_a)
    dodeg(cidx_a)
    sstart(rows_a, cidx_a, sems_a)
    swait(rows_a, cidx_a, sems_a)

    pltpu.sync_copy(deg_v, degall_out.at[pl.ds((c * 16 + s) * NP, NP)])
    plsc.subcore_barrier()

    pltpu.sync_copy(degall_out.at[pl.ds(c * 16 * NP + s * RPT, RPT)], acc_v)

    def red(t, carry):
        pltpu.sync_copy(degall_out.at[pl.ds((c * 16 + t) * NP + s * RPT, RPT)],
                        tmp_v)
        for k in range(RPT // 16):
            sl = pl.ds(k * 16, 16)
            acc_v[sl] = acc_v[sl] + tmp_v[sl]
        return carry

    lax.fori_loop(1, 16, red, 0)

    pltpu.sync_copy(agg_sh.at[pl.ds(s * RPT, RPT)],
                    agg_out.at[c, pl.ds(s * RPT, RPT)])
    pltpu.sync_copy(acc_v, deg_out.at[c, pl.ds(s * RPT, RPT)])


def _stage2(msg, row, col, zeros):
    kern = functools.partial(
        pl.kernel,
        mesh=plsc.VectorSubcoreMesh(core_axis_name="c", subcore_axis_name="s"),
        out_type=(
            jax.ShapeDtypeStruct((2, NP, D), jnp.float32),
            jax.ShapeDtypeStruct((2, NP), jnp.float32),
            jax.ShapeDtypeStruct((2 * 16 * NP,), jnp.float32),
        ),
        scratch_types=[
            pltpu.VMEM((EPW,), jnp.int32),
            pltpu.VMEM((K,), jnp.int32),
            pltpu.VMEM((K,), jnp.int32),
            pltpu.VMEM((K, D), jnp.float32),
            pltpu.VMEM((K, D), jnp.float32),
            pltpu.VMEM((NP,), jnp.float32),
            pltpu.VMEM((RPT,), jnp.float32),
            pltpu.VMEM((RPT,), jnp.float32),
            pltpu.VMEM_SHARED((NP, D), jnp.float32),
            pltpu.SemaphoreType.DMA,
            pltpu.SemaphoreType.DMA,
            pltpu.SemaphoreType.DMA,
            pltpu.SemaphoreType.DMA,
            pltpu.SemaphoreType.DMA,
            pltpu.SemaphoreType.DMA,
        ],
        compiler_params=pltpu.CompilerParams(needs_layout_passes=False),
    )(_sc_scatter_body)
    return kern(msg, row, col, zeros)


def _s3_body(x_ref, a0_ref, a1_ref, deg_ref, wf_ref, bf_ref, wo1a_ref,
             wo1b_ref, bo1_ref, wo2_ref, bo2_ref, o_ref):
    s = a0_ref[...] + a1_ref[...]
    aggn = s / jnp.maximum(deg_ref[...], 1.0)
    fx = x_ref[...] + jnp.tanh(aggn @ wf_ref[...] + bf_ref[...])
    o = jnp.maximum(fx @ wo1a_ref[...] + aggn @ wo1b_ref[...] + bo1_ref[...], 0.0)
    o_ref[...] = o @ wo2_ref[...] + bo2_ref[...]


def _stage3(x_pad, a0, a1, degb, Wf, bf, Wo1, bo1, Wo2, bo2):
    grid = NP // S3_BLK
    return pl.pallas_call(
        _s3_body,
        grid=(grid,),
        in_specs=[
            pl.BlockSpec((S3_BLK, D), lambda i: (i, 0)),
            pl.BlockSpec((S3_BLK, D), lambda i: (i, 0)),
            pl.BlockSpec((S3_BLK, D), lambda i: (i, 0)),
            pl.BlockSpec((S3_BLK, D), lambda i: (i, 0)),
            pl.BlockSpec((D, D), lambda i: (0, 0)),
            pl.BlockSpec((1, D), lambda i: (0, 0)),
            pl.BlockSpec((D, D), lambda i: (0, 0)),
            pl.BlockSpec((D, D), lambda i: (0, 0)),
            pl.BlockSpec((1, D), lambda i: (0, 0)),
            pl.BlockSpec((D, OUT), lambda i: (0, 0)),
            pl.BlockSpec((1, OUT), lambda i: (0, 0)),
        ],
        out_specs=pl.BlockSpec((S3_BLK, OUT), lambda i: (i, 0)),
        out_shape=jax.ShapeDtypeStruct((NP, OUT), jnp.float32),
    )(x_pad, a0, a1, degb, Wf, bf.reshape(1, D), Wo1[:D], Wo1[D:],
      bo1.reshape(1, D), Wo2, bo2.reshape(1, OUT))


def kernel(x, edge_index, batch, Wm1, bm1, Wm2, bm2, Wf, bf, Wo1, bo1, Wo2, bo2):
    x_pad = jnp.zeros((NP, D), jnp.float32).at[:N].set(x)
    msg = _stage1(x_pad, Wm1, bm1, Wm2, bm2)
    row = edge_index[0]
    col = edge_index[1]
    zeros = jnp.zeros((NP, D), jnp.float32)
    agg, deg, _ = _stage2(msg, row, col, zeros)
    degb = jnp.broadcast_to((deg[0] + deg[1]).reshape(NP, 1), (NP, D))
    out = _stage3(x_pad, agg[0], agg[1], degb, Wf, bf, Wo1, bo1, Wo2, bo2)
    return out[:N]


# speedup vs baseline: 8.9642x; 1.0014x over previous
import functools

import jax
import jax.numpy as jnp
from jax import lax
from jax.experimental import pallas as pl
from jax.experimental.pallas import tpu as pltpu
from jax.experimental.pallas import tpu_sc as plsc

N = 10000
E = 320000
D = 128
OUT = 128

NP = 10240
DR = NP // D
NW = 32
EPW = E // NW
K = 80
CHUNKS = EPW // K
RPT = NP // 16

S1_BLK = 1024
S3_BLK = 1024


def _s1_body(x_ref, wm1_ref, bm1_ref, wm2_ref, bm2_ref, o_ref):
    h = jnp.maximum(x_ref[...] @ wm1_ref[...] + bm1_ref[...], 0.0)
    o_ref[...] = h @ wm2_ref[...] + bm2_ref[...]


def _stage1(x_pad, Wm1, bm1, Wm2, bm2):
    grid = NP // S1_BLK
    return pl.pallas_call(
        _s1_body,
        grid=(grid,),
        in_specs=[
            pl.BlockSpec((S1_BLK, D), lambda i: (i, 0)),
            pl.BlockSpec((D, D), lambda i: (0, 0)),
            pl.BlockSpec((1, D), lambda i: (0, 0)),
            pl.BlockSpec((D, D), lambda i: (0, 0)),
            pl.BlockSpec((1, D), lambda i: (0, 0)),
        ],
        out_specs=pl.BlockSpec((S1_BLK, D), lambda i: (i, 0)),
        out_shape=jax.ShapeDtypeStruct((NP, D), jnp.float32),
    )(x_pad, Wm1, bm1.reshape(1, D), Wm2, bm2.reshape(1, D))


def _sc_scatter_body(msg_hbm, row_hbm, col_hbm, zeros_hbm,
                     agg_out, deg_out, degall_out,
                     ridx0, ridx1, ridx2, cidx0, cidx1, cidx2,
                     rows0, rows1, rows2, deg_v, tmp_v, acc_v, agg_sh,
                     semg0, semg1, semg2, sems0, sems1, sems2,
                     semi0, semi1, semi2):
    c = lax.axis_index("c")
    s = lax.axis_index("s")
    wid = s * 2 + c
    base = wid * EPW

    RIDX = (ridx0, ridx1, ridx2)
    CIDX = (cidx0, cidx1, cidx2)
    ROWS = (rows0, rows1, rows2)
    SEMG = (semg0, semg1, semg2)
    SEMS = (sems0, sems1, sems2)
    SEMI = (semi0, semi1, semi2)

    pltpu.sync_copy(zeros_hbm.at[pl.ds(s * RPT, RPT)],
                    agg_sh.at[pl.ds(s * RPT, RPT)])

    def zinit(i, carry):
        deg_v[pl.ds(i * 16, 16)] = jnp.zeros((16,), jnp.float32)
        return carry

    lax.fori_loop(0, NP // 16, zinit, 0)

    plsc.subcore_barrier()

    ones = jnp.full((16,), 1.0, jnp.float32)

    def i2start(i, p):
        pltpu.async_copy(row_hbm.at[pl.ds(base + i * K, K)], RIDX[p], SEMI[p])
        pltpu.async_copy(col_hbm.at[pl.ds(base + i * K, K)], CIDX[p], SEMI[p])

    def i2wait(p):
        pltpu.make_async_copy(row_hbm.at[pl.ds(0, K)], RIDX[p], SEMI[p]).wait()
        pltpu.make_async_copy(col_hbm.at[pl.ds(0, K)], CIDX[p], SEMI[p]).wait()

    def gstart(i, p):
        pltpu.async_copy(msg_hbm.at[RIDX[p]], ROWS[p], SEMG[p])

    def gwait(b):
        pltpu.make_async_copy(msg_hbm.at[RIDX[b]], ROWS[b], SEMG[b]).wait()

    def sstart(i, b):
        pltpu.async_copy(ROWS[b], agg_sh.at[CIDX[b]], SEMS[b], add=True)

    def swait(p):
        pltpu.make_async_copy(ROWS[p], agg_sh.at[CIDX[p]], SEMS[p]).wait()

    def dodeg(b):
        for j in range(K // 16):
            cc = CIDX[b][pl.ds(j * 16, 16)]
            plsc.addupdate_scatter(deg_v, [cc], ones)

    def full_step(i, b, p, first=False):
        if not first:
            swait(p)
        i2start(i + 1, p)
        gwait(b)
        dodeg(b)
        sstart(i, b)
        i2wait(p)
        gstart(i + 1, p)

    pltpu.sync_copy(row_hbm.at[pl.ds(base, K)], ridx0)
    pltpu.sync_copy(col_hbm.at[pl.ds(base, K)], cidx0)
    gstart(0, 0)
    full_step(0, 0, 1, first=True)
    full_step(1, 1, 2, first=True)

    def triple(q, carry):
        i0 = 3 * q + 2
        full_step(i0, 2, 0)
        full_step(i0 + 1, 0, 1)
        full_step(i0 + 2, 1, 2)
        return carry

    lax.fori_loop(0, (CHUNKS - 5) // 3, triple, 0)

    full_step(CHUNKS - 3, 2, 0)
    full_step(CHUNKS - 2, 0, 1)
    swait(2)
    gwait(1)
    dodeg(1)
    sstart(CHUNKS - 1, 1)
    swait(0)
    swait(1)

    pltpu.sync_copy(deg_v, degall_out.at[pl.ds((c * 16 + s) * NP, NP)])
    plsc.subcore_barrier()

    pltpu.sync_copy(degall_out.at[pl.ds(c * 16 * NP + s * RPT, RPT)], acc_v)

    def red(t, carry):
        pltpu.sync_copy(degall_out.at[pl.ds((c * 16 + t) * NP + s * RPT, RPT)],
                        tmp_v)
        for k in range(RPT // 16):
            sl = pl.ds(k * 16, 16)
            acc_v[sl] = acc_v[sl] + tmp_v[sl]
        return carry

    lax.fori_loop(1, 16, red, 0)

    pltpu.sync_copy(agg_sh.at[pl.ds(s * RPT, RPT)],
                    agg_out.at[c, pl.ds(s * RPT, RPT)])
    pltpu.sync_copy(acc_v, deg_out.at[c, pl.ds(s * RPT, RPT)])


def _stage2(msg, row, col, zeros):
    kern = functools.partial(
        pl.kernel,
        mesh=plsc.VectorSubcoreMesh(core_axis_name="c", subcore_axis_name="s"),
        out_type=(
            jax.ShapeDtypeStruct((2, NP, D), jnp.float32),
            jax.ShapeDtypeStruct((2, NP), jnp.float32),
            jax.ShapeDtypeStruct((2 * 16 * NP,), jnp.float32),
        ),
        scratch_types=[
            pltpu.VMEM((K,), jnp.int32),
            pltpu.VMEM((K,), jnp.int32),
            pltpu.VMEM((K,), jnp.int32),
            pltpu.VMEM((K,), jnp.int32),
            pltpu.VMEM((K,), jnp.int32),
            pltpu.VMEM((K,), jnp.int32),
            pltpu.VMEM((K, D), jnp.float32),
            pltpu.VMEM((K, D), jnp.float32),
            pltpu.VMEM((K, D), jnp.float32),
            pltpu.VMEM((NP,), jnp.float32),
            pltpu.VMEM((RPT,), jnp.float32),
            pltpu.VMEM((RPT,), jnp.float32),
            pltpu.VMEM_SHARED((NP, D), jnp.float32),
        ] + [pltpu.SemaphoreType.DMA] * 9,
        compiler_params=pltpu.CompilerParams(needs_layout_passes=False),
    )(_sc_scatter_body)
    return kern(msg, row, col, zeros)


def _s3_body(x_ref, a0_ref, a1_ref, deg_ref, wf_ref, bf_ref, wo1a_ref,
             wo1b_ref, bo1_ref, wo2_ref, bo2_ref, o_ref):
    s = a0_ref[...] + a1_ref[...]
    aggn = s / jnp.maximum(deg_ref[...], 1.0)
    fx = x_ref[...] + jnp.tanh(aggn @ wf_ref[...] + bf_ref[...])
    o = jnp.maximum(fx @ wo1a_ref[...] + aggn @ wo1b_ref[...] + bo1_ref[...], 0.0)
    o_ref[...] = o @ wo2_ref[...] + bo2_ref[...]


def _stage3(x_pad, a0, a1, degb, Wf, bf, Wo1, bo1, Wo2, bo2):
    grid = NP // S3_BLK
    return pl.pallas_call(
        _s3_body,
        grid=(grid,),
        in_specs=[
            pl.BlockSpec((S3_BLK, D), lambda i: (i, 0)),
            pl.BlockSpec((S3_BLK, D), lambda i: (i, 0)),
            pl.BlockSpec((S3_BLK, D), lambda i: (i, 0)),
            pl.BlockSpec((S3_BLK, D), lambda i: (i, 0)),
            pl.BlockSpec((D, D), lambda i: (0, 0)),
            pl.BlockSpec((1, D), lambda i: (0, 0)),
            pl.BlockSpec((D, D), lambda i: (0, 0)),
            pl.BlockSpec((D, D), lambda i: (0, 0)),
            pl.BlockSpec((1, D), lambda i: (0, 0)),
            pl.BlockSpec((D, OUT), lambda i: (0, 0)),
            pl.BlockSpec((1, OUT), lambda i: (0, 0)),
        ],
        out_specs=pl.BlockSpec((S3_BLK, OUT), lambda i: (i, 0)),
        out_shape=jax.ShapeDtypeStruct((NP, OUT), jnp.float32),
    )(x_pad, a0, a1, degb, Wf, bf.reshape(1, D), Wo1[:D], Wo1[D:],
      bo1.reshape(1, D), Wo2, bo2.reshape(1, OUT))


def kernel(x, edge_index, batch, Wm1, bm1, Wm2, bm2, Wf, bf, Wo1, bo1, Wo2, bo2):
    x_pad = jnp.zeros((NP, D), jnp.float32).at[:N].set(x)
    msg = _stage1(x_pad, Wm1, bm1, Wm2, bm2)
    row = edge_index[0]
    col = edge_index[1]
    zeros = jnp.zeros((NP, D), jnp.float32)
    agg, deg, _ = _stage2(msg, row, col, zeros)
    degb = jnp.broadcast_to((deg[0] + deg[1]).reshape(NP, 1), (NP, D))
    out = _stage3(x_pad, agg[0], agg[1], degb, Wf, bf, Wo1, bo1, Wo2, bo2)
    return out[:N]


# speedup vs baseline: 10.5190x; 1.1735x over previous
import functools

import jax
import jax.numpy as jnp
from jax import lax
from jax.experimental import pallas as pl
from jax.experimental.pallas import tpu as pltpu
from jax.experimental.pallas import tpu_sc as plsc

N = 10000
E = 320000
D = 128
OUT = 128

NP = 10240
NW = 32
K = 128
CHUNKS = 79
EPW = CHUNKS * K
EP = NW * EPW
RPT = NP // 16

S1_BLK = 1000
S3_BLK = 1000


def _s1_body(x_ref, wm1_ref, bm1_ref, wm2_ref, bm2_ref, o_ref):
    h = jnp.maximum(x_ref[...] @ wm1_ref[...] + bm1_ref[...], 0.0)
    o_ref[...] = h @ wm2_ref[...] + bm2_ref[...]


def _stage1(x, Wm1, bm1, Wm2, bm2):
    grid = N // S1_BLK
    return pl.pallas_call(
        _s1_body,
        grid=(grid,),
        in_specs=[
            pl.BlockSpec((S1_BLK, D), lambda i: (i, 0)),
            pl.BlockSpec((D, D), lambda i: (0, 0)),
            pl.BlockSpec((1, D), lambda i: (0, 0)),
            pl.BlockSpec((D, D), lambda i: (0, 0)),
            pl.BlockSpec((1, D), lambda i: (0, 0)),
        ],
        out_specs=pl.BlockSpec((S1_BLK, D), lambda i: (i, 0)),
        out_shape=jax.ShapeDtypeStruct((N, D), jnp.float32),
    )(x, Wm1, bm1.reshape(1, D), Wm2, bm2.reshape(1, D))


def _sc_scatter_body(msg_hbm, row_hbm, col_hbm, zeros_hbm,
                     agg_out, deg_out, degall_out,
                     ridx0, ridx1, cidx0, cidx1,
                     rows0, rows1, deg_v, tmp_v, acc_v, agg_sh,
                     semg0, semg1, sems0, sems1, semi0, semi1):
    c = lax.axis_index("c")
    s = lax.axis_index("s")
    wid = s * 2 + c
    base = wid * EPW

    RIDX = (ridx0, ridx1)
    CIDX = (cidx0, cidx1)
    ROWS = (rows0, rows1)
    SEMG = (semg0, semg1)
    SEMS = (sems0, sems1)
    SEMI = (semi0, semi1)

    pltpu.sync_copy(zeros_hbm.at[pl.ds(s * RPT, RPT)],
                    agg_sh.at[pl.ds(s * RPT, RPT)])

    def zinit(i, carry):
        deg_v[pl.ds(i * 16, 16)] = jnp.zeros((16,), jnp.float32)
        return carry

    lax.fori_loop(0, NP // 16, zinit, 0)

    plsc.subcore_barrier()

    ones = jnp.full((16,), 1.0, jnp.float32)

    def i2start(i, p):
        pltpu.async_copy(row_hbm.at[pl.ds(base + i * K, K)], RIDX[p], SEMI[p])
        pltpu.async_copy(col_hbm.at[pl.ds(base + i * K, K)], CIDX[p], SEMI[p])

    def i2wait(p):
        pltpu.make_async_copy(row_hbm.at[pl.ds(0, K)], RIDX[p], SEMI[p]).wait()
        pltpu.make_async_copy(col_hbm.at[pl.ds(0, K)], CIDX[p], SEMI[p]).wait()

    def gstart(i, p):
        pltpu.async_copy(msg_hbm.at[RIDX[p]], ROWS[p], SEMG[p])

    def gwait(b):
        pltpu.make_async_copy(msg_hbm.at[RIDX[b]], ROWS[b], SEMG[b]).wait()

    def sstart(i, b):
        pltpu.async_copy(ROWS[b], agg_sh.at[CIDX[b]], SEMS[b], add=True)

    def swait(p):
        pltpu.make_async_copy(ROWS[p], agg_sh.at[CIDX[p]], SEMS[p]).wait()

    def dodeg(b):
        for j in range(K // 16):
            cc = CIDX[b][pl.ds(j * 16, 16)]
            plsc.addupdate_scatter(deg_v, [cc], ones)

    def full_step(i, b, p, first=False):
        if not first:
            swait(p)
        i2start(i + 1, p)
        gwait(b)
        dodeg(b)
        sstart(i, b)
        i2wait(p)
        gstart(i + 1, p)

    pltpu.sync_copy(row_hbm.at[pl.ds(base, K)], ridx0)
    pltpu.sync_copy(col_hbm.at[pl.ds(base, K)], cidx0)
    gstart(0, 0)
    full_step(0, 0, 1, first=True)

    def pair(q, carry):
        i0 = 2 * q + 1
        full_step(i0, 1, 0)
        full_step(i0 + 1, 0, 1)
        return carry

    lax.fori_loop(0, (CHUNKS - 3) // 2, pair, 0)

    full_step(CHUNKS - 2, 1, 0)
    swait(1)
    gwait(0)
    dodeg(0)
    sstart(CHUNKS - 1, 0)
    swait(0)

    pltpu.sync_copy(deg_v, degall_out.at[pl.ds((c * 16 + s) * NP, NP)])
    plsc.subcore_barrier()

    pltpu.sync_copy(degall_out.at[pl.ds(c * 16 * NP + s * RPT, RPT)], acc_v)

    def red(t, carry):
        pltpu.sync_copy(degall_out.at[pl.ds((c * 16 + t) * NP + s * RPT, RPT)],
                        tmp_v)
        for k in range(RPT // 16):
            sl = pl.ds(k * 16, 16)
            acc_v[sl] = acc_v[sl] + tmp_v[sl]
        return carry

    lax.fori_loop(1, 16, red, 0)

    pltpu.sync_copy(agg_sh.at[pl.ds(s * RPT, RPT)],
                    agg_out.at[c, pl.ds(s * RPT, RPT)])
    pltpu.sync_copy(acc_v, deg_out.at[c, pl.ds(s * RPT, RPT)])


def _stage2(msg, row, col, zeros):
    kern = functools.partial(
        pl.kernel,
        mesh=plsc.VectorSubcoreMesh(core_axis_name="c", subcore_axis_name="s"),
        out_type=(
            jax.ShapeDtypeStruct((2, NP, D), jnp.float32),
            jax.ShapeDtypeStruct((2, NP), jnp.float32),
            jax.ShapeDtypeStruct((2 * 16 * NP,), jnp.float32),
        ),
        scratch_types=[
            pltpu.VMEM((K,), jnp.int32),
            pltpu.VMEM((K,), jnp.int32),
            pltpu.VMEM((K,), jnp.int32),
            pltpu.VMEM((K,), jnp.int32),
            pltpu.VMEM((K, D), jnp.float32),
            pltpu.VMEM((K, D), jnp.float32),
            pltpu.VMEM((NP,), jnp.float32),
            pltpu.VMEM((RPT,), jnp.float32),
            pltpu.VMEM((RPT,), jnp.float32),
            pltpu.VMEM_SHARED((NP, D), jnp.float32),
        ] + [pltpu.SemaphoreType.DMA] * 6,
        compiler_params=pltpu.CompilerParams(needs_layout_passes=False),
    )(_sc_scatter_body)
    return kern(msg, row, col, zeros)


def _s3_body(x_ref, a0_ref, a1_ref, deg_ref, wf_ref, bf_ref, wo1a_ref,
             wo1b_ref, bo1_ref, wo2_ref, bo2_ref, o_ref):
    s = a0_ref[0] + a1_ref[0]
    aggn = s / jnp.maximum(deg_ref[...], 1.0)
    fx = x_ref[...] + jnp.tanh(aggn @ wf_ref[...] + bf_ref[...])
    o = jnp.maximum(fx @ wo1a_ref[...] + aggn @ wo1b_ref[...] + bo1_ref[...], 0.0)
    o_ref[...] = o @ wo2_ref[...] + bo2_ref[...]


def _stage3(x, agg, degb, Wf, bf, Wo1, bo1, Wo2, bo2):
    grid = N // S3_BLK
    return pl.pallas_call(
        _s3_body,
        grid=(grid,),
        in_specs=[
            pl.BlockSpec((S3_BLK, D), lambda i: (i, 0)),
            pl.BlockSpec((1, S3_BLK, D), lambda i: (0, i, 0)),
            pl.BlockSpec((1, S3_BLK, D), lambda i: (1, i, 0)),
            pl.BlockSpec((S3_BLK, D), lambda i: (i, 0)),
            pl.BlockSpec((D, D), lambda i: (0, 0)),
            pl.BlockSpec((1, D), lambda i: (0, 0)),
            pl.BlockSpec((D, D), lambda i: (0, 0)),
            pl.BlockSpec((D, D), lambda i: (0, 0)),
            pl.BlockSpec((1, D), lambda i: (0, 0)),
            pl.BlockSpec((D, OUT), lambda i: (0, 0)),
            pl.BlockSpec((1, OUT), lambda i: (0, 0)),
        ],
        out_specs=pl.BlockSpec((S3_BLK, OUT), lambda i: (i, 0)),
        out_shape=jax.ShapeDtypeStruct((N, OUT), jnp.float32),
    )(x, agg, agg, degb, Wf, bf.reshape(1, D), Wo1[:D], Wo1[D:],
      bo1.reshape(1, D), Wo2, bo2.reshape(1, OUT))


def kernel(x, edge_index, batch, Wm1, bm1, Wm2, bm2, Wf, bf, Wo1, bo1, Wo2, bo2):
    msg = _stage1(x, Wm1, bm1, Wm2, bm2)
    pad = EP - E
    pr = jnp.arange(pad, dtype=jnp.int32) % N
    pc = N + jnp.arange(pad, dtype=jnp.int32) % (NP - N)
    row = jnp.concatenate([edge_index[0], pr])
    col = jnp.concatenate([edge_index[1], pc])
    zeros = jnp.zeros((NP, D), jnp.float32)
    agg, deg, _ = _stage2(msg, row, col, zeros)
    degb = jnp.broadcast_to((deg[0, :N] + deg[1, :N]).reshape(N, 1), (N, D))
    return _stage3(x, agg, degb, Wf, bf, Wo1, bo1, Wo2, bo2)


# speedup vs baseline: 10.7908x; 1.0258x over previous
import functools

import jax
import jax.numpy as jnp
from jax import lax
from jax.experimental import pallas as pl
from jax.experimental.pallas import tpu as pltpu
from jax.experimental.pallas import tpu_sc as plsc

N = 10000
E = 320000
D = 128
OUT = 128

NP = 10240
NW = 32
K = 128
CHUNKS = 79
EPW = CHUNKS * K
LAST_CHUNKS = (E - (NW - 1) * EPW) // K
RPT = NP // 16

S1_BLK = 1000
SZ_BLK = 1024
S3_BLK = 1000


def _s1_body(x_ref, wm1_ref, bm1_ref, wm2_ref, bm2_ref, o_ref, z_ref):
    h = jnp.maximum(x_ref[...] @ wm1_ref[...] + bm1_ref[...], 0.0)
    o_ref[...] = h @ wm2_ref[...] + bm2_ref[...]
    z_ref[...] = jnp.zeros((SZ_BLK, D), jnp.float32)


def _stage1(x, Wm1, bm1, Wm2, bm2):
    grid = N // S1_BLK
    return pl.pallas_call(
        _s1_body,
        grid=(grid,),
        in_specs=[
            pl.BlockSpec((S1_BLK, D), lambda i: (i, 0)),
            pl.BlockSpec((D, D), lambda i: (0, 0)),
            pl.BlockSpec((1, D), lambda i: (0, 0)),
            pl.BlockSpec((D, D), lambda i: (0, 0)),
            pl.BlockSpec((1, D), lambda i: (0, 0)),
        ],
        out_specs=[
            pl.BlockSpec((S1_BLK, D), lambda i: (i, 0)),
            pl.BlockSpec((SZ_BLK, D), lambda i: (i, 0)),
        ],
        out_shape=[
            jax.ShapeDtypeStruct((N, D), jnp.float32),
            jax.ShapeDtypeStruct((NP, D), jnp.float32),
        ],
    )(x, Wm1, bm1.reshape(1, D), Wm2, bm2.reshape(1, D))


def _sc_scatter_body(msg_hbm, row_hbm, col_hbm, zeros_hbm,
                     agg_out, deg_out, degall_out,
                     ridx0, ridx1, cidx0, cidx1, rows0, rows1,
                     deg_v, tmp0_v, tmp1_v, acc_v, agg_sh,
                     semg0, semg1, sems0, sems1, semi0, semi1):
    c = lax.axis_index("c")
    s = lax.axis_index("s")
    wid = s * 2 + c
    base = wid * EPW
    nc = jnp.where(wid == NW - 1, LAST_CHUNKS, CHUNKS)

    RIDX = (ridx0, ridx1)
    CIDX = (cidx0, cidx1)
    ROWS = (rows0, rows1)
    SEMG = (semg0, semg1)
    SEMS = (sems0, sems1)
    SEMI = (semi0, semi1)
    TMP = (tmp0_v, tmp1_v)

    pltpu.sync_copy(zeros_hbm.at[pl.ds(s * RPT, RPT)],
                    agg_sh.at[pl.ds(s * RPT, RPT)])

    def zinit(i, carry):
        deg_v[pl.ds(i * 16, 16)] = jnp.zeros((16,), jnp.float32)
        return carry

    lax.fori_loop(0, NP // 16, zinit, 0)

    plsc.subcore_barrier()

    ones = jnp.full((16,), 1.0, jnp.float32)

    def i2start(i, p):
        pltpu.async_copy(row_hbm.at[pl.ds(base + i * K, K)], RIDX[p], SEMI[p])
        pltpu.async_copy(col_hbm.at[pl.ds(base + i * K, K)], CIDX[p], SEMI[p])

    def i2wait(p):
        pltpu.make_async_copy(row_hbm.at[pl.ds(0, K)], RIDX[p], SEMI[p]).wait()
        pltpu.make_async_copy(col_hbm.at[pl.ds(0, K)], CIDX[p], SEMI[p]).wait()

    def gstart(i, p):
        pltpu.async_copy(msg_hbm.at[RIDX[p]], ROWS[p], SEMG[p])

    def gwait(b):
        pltpu.make_async_copy(msg_hbm.at[RIDX[b]], ROWS[b], SEMG[b]).wait()

    def sstart(i, b):
        pltpu.async_copy(ROWS[b], agg_sh.at[CIDX[b]], SEMS[b], add=True)

    def swait(p):
        pltpu.make_async_copy(ROWS[p], agg_sh.at[CIDX[p]], SEMS[p]).wait()

    def dodeg(b):
        for j in range(K // 16):
            cc = CIDX[b][pl.ds(j * 16, 16)]
            plsc.addupdate_scatter(deg_v, [cc], ones)

    def full_step(i, b, p, first=False):
        if not first:
            swait(p)
        i2start(i + 1, p)
        gwait(b)
        dodeg(b)
        sstart(i, b)
        i2wait(p)
        gstart(i + 1, p)

    pltpu.sync_copy(row_hbm.at[pl.ds(base, K)], ridx0)
    pltpu.sync_copy(col_hbm.at[pl.ds(base, K)], cidx0)
    gstart(0, 0)
    full_step(0, 0, 1, first=True)

    def pair(q, carry):
        i0 = 2 * q + 1
        full_step(i0, 1, 0)
        full_step(i0 + 1, 0, 1)
        return carry

    lax.fori_loop(0, (nc - 3) // 2, pair, 0)

    swait(0)
    i2start(nc - 1, 0)
    gwait(1)
    dodeg(1)
    sstart(nc - 2, 1)
    i2wait(0)
    gstart(nc - 1, 0)
    swait(1)
    gwait(0)
    dodeg(0)
    sstart(nc - 1, 0)
    swait(0)

    pltpu.sync_copy(deg_v, degall_out.at[pl.ds((c * 16 + s) * NP, NP)])
    plsc.subcore_barrier()

    def rstart(t, u):
        pltpu.async_copy(
            degall_out.at[pl.ds((c * 16 + t) * NP + s * RPT, RPT)],
            TMP[u], SEMI[u])

    def rwait(u):
        pltpu.make_async_copy(degall_out.at[pl.ds(0, RPT)], TMP[u],
                              SEMI[u]).wait()

    pltpu.sync_copy(degall_out.at[pl.ds(c * 16 * NP + s * RPT, RPT)], acc_v)
    rstart(1, 1)
    for t in range(1, 16):
        if t < 15:
            rstart(t + 1, (t + 1) % 2)
        rwait(t % 2)
        for k in range(RPT // 16):
            sl = pl.ds(k * 16, 16)
            acc_v[sl] = acc_v[sl] + TMP[t % 2][sl]

    pltpu.sync_copy(acc_v, deg_out.at[c, pl.ds(s * RPT, RPT)])

    pltpu.sync_copy(agg_sh.at[pl.ds(s * RPT, RPT)],
                    agg_out.at[c, pl.ds(s * RPT, RPT)])


def _stage2(msg, row, col, zeros):
    kern = functools.partial(
        pl.kernel,
        mesh=plsc.VectorSubcoreMesh(core_axis_name="c", subcore_axis_name="s"),
        out_type=(
            jax.ShapeDtypeStruct((2, NP, D), jnp.float32),
            jax.ShapeDtypeStruct((2, NP), jnp.float32),
            jax.ShapeDtypeStruct((2 * 16 * NP,), jnp.float32),
        ),
        scratch_types=[
            pltpu.VMEM((K,), jnp.int32),
            pltpu.VMEM((K,), jnp.int32),
            pltpu.VMEM((K,), jnp.int32),
            pltpu.VMEM((K,), jnp.int32),
            pltpu.VMEM((K, D), jnp.float32),
            pltpu.VMEM((K, D), jnp.float32),
            pltpu.VMEM((NP,), jnp.float32),
            pltpu.VMEM((RPT,), jnp.float32),
            pltpu.VMEM((RPT,), jnp.float32),
            pltpu.VMEM((RPT,), jnp.float32),
            pltpu.VMEM_SHARED((NP, D), jnp.float32),
        ] + [pltpu.SemaphoreType.DMA] * 6,
        compiler_params=pltpu.CompilerParams(needs_layout_passes=False),
    )(_sc_scatter_body)
    return kern(msg, row, col, zeros)


def _s3_body(x_ref, a0_ref, a1_ref, deg_ref, wf_ref, bf_ref, wo1a_ref,
             wo1b_ref, bo1_ref, wo2_ref, bo2_ref, o_ref):
    s = a0_ref[0] + a1_ref[0]
    aggn = s / jnp.maximum(deg_ref[...], 1.0)
    fx = x_ref[...] + jnp.tanh(aggn @ wf_ref[...] + bf_ref[...])
    o = jnp.maximum(fx @ wo1a_ref[...] + aggn @ wo1b_ref[...] + bo1_ref[...], 0.0)
    o_ref[...] = o @ wo2_ref[...] + bo2_ref[...]


def _stage3(x, agg, degb, Wf, bf, Wo1, bo1, Wo2, bo2):
    grid = N // S3_BLK
    return pl.pallas_call(
        _s3_body,
        grid=(grid,),
        in_specs=[
            pl.BlockSpec((S3_BLK, D), lambda i: (i, 0)),
            pl.BlockSpec((1, S3_BLK, D), lambda i: (0, i, 0)),
            pl.BlockSpec((1, S3_BLK, D), lambda i: (1, i, 0)),
            pl.BlockSpec((S3_BLK, D), lambda i: (i, 0)),
            pl.BlockSpec((D, D), lambda i: (0, 0)),
            pl.BlockSpec((1, D), lambda i: (0, 0)),
            pl.BlockSpec((D, D), lambda i: (0, 0)),
            pl.BlockSpec((D, D), lambda i: (0, 0)),
            pl.BlockSpec((1, D), lambda i: (0, 0)),
            pl.BlockSpec((D, OUT), lambda i: (0, 0)),
            pl.BlockSpec((1, OUT), lambda i: (0, 0)),
        ],
        out_specs=pl.BlockSpec((S3_BLK, OUT), lambda i: (i, 0)),
        out_shape=jax.ShapeDtypeStruct((N, OUT), jnp.float32),
    )(x, agg, agg, degb, Wf, bf.reshape(1, D), Wo1[:D], Wo1[D:],
      bo1.reshape(1, D), Wo2, bo2.reshape(1, OUT))


def kernel(x, edge_index, batch, Wm1, bm1, Wm2, bm2, Wf, bf, Wo1, bo1, Wo2, bo2):
    msg, zeros = _stage1(x, Wm1, bm1, Wm2, bm2)
    agg, deg, _ = _stage2(msg, edge_index[0], edge_index[1], zeros)
    degb = jnp.broadcast_to((deg[0, :N] + deg[1, :N]).reshape(N, 1), (N, D))
    return _stage3(x, agg, degb, Wf, bf, Wo1, bo1, Wo2, bo2)


# speedup vs baseline: 11.0942x; 1.0281x over previous
import functools

import jax
import jax.numpy as jnp
from jax import lax
from jax.experimental import pallas as pl
from jax.experimental.pallas import tpu as pltpu
from jax.experimental.pallas import tpu_sc as plsc

N = 10000
E = 320000
D = 128
OUT = 128

NP = 10240
NW = 32
K = 128
CHUNKS = 79
EPW = CHUNKS * K
LAST_CHUNKS = (E - (NW - 1) * EPW) // K
RPT = NP // 16

S1_BLK = 1000
SZ_BLK = 1024
S3_BLK = 1000


def _s1_body(x_ref, wm1_ref, bm1_ref, wm2_ref, bm2_ref, o_ref, z_ref):
    h = jnp.maximum(x_ref[...] @ wm1_ref[...] + bm1_ref[...], 0.0)
    o_ref[...] = h @ wm2_ref[...] + bm2_ref[...]
    z_ref[...] = jnp.zeros((SZ_BLK, D), jnp.float32)


def _stage1(x, Wm1, bm1, Wm2, bm2):
    grid = N // S1_BLK
    return pl.pallas_call(
        _s1_body,
        grid=(grid,),
        in_specs=[
            pl.BlockSpec((S1_BLK, D), lambda i: (i, 0)),
            pl.BlockSpec((D, D), lambda i: (0, 0)),
            pl.BlockSpec((1, D), lambda i: (0, 0)),
            pl.BlockSpec((D, D), lambda i: (0, 0)),
            pl.BlockSpec((1, D), lambda i: (0, 0)),
        ],
        out_specs=[
            pl.BlockSpec((S1_BLK, D), lambda i: (i, 0)),
            pl.BlockSpec((SZ_BLK, D), lambda i: (i, 0)),
        ],
        out_shape=[
            jax.ShapeDtypeStruct((N, D), jnp.float32),
            jax.ShapeDtypeStruct((NP, D), jnp.float32),
        ],
    )(x, Wm1, bm1.reshape(1, D), Wm2, bm2.reshape(1, D))


def _sc_scatter_body(msg_hbm, row_hbm, col_hbm, zeros_hbm,
                     agg_out, deg_out, degall_out,
                     ridx0, ridx1, cidx0, cidx1, rows0, rows1,
                     deg_v, tmp0_v, tmp1_v, acc_v, agg_sh,
                     semg0, semg1, sems0, sems1, semi0, semi1):
    c = lax.axis_index("c")
    s = lax.axis_index("s")
    wid = s * 2 + c
    base = wid * EPW
    nc = jnp.where(wid == NW - 1, LAST_CHUNKS, CHUNKS)

    RIDX = (ridx0, ridx1)
    CIDX = (cidx0, cidx1)
    ROWS = (rows0, rows1)
    SEMG = (semg0, semg1)
    SEMS = (sems0, sems1)
    SEMI = (semi0, semi1)
    TMP = (tmp0_v, tmp1_v)

    pltpu.async_copy(zeros_hbm.at[pl.ds(s * RPT, RPT)],
                     agg_sh.at[pl.ds(s * RPT, RPT)], semg1)

    def zinit(i, carry):
        deg_v[pl.ds(i * 16, 16)] = jnp.zeros((16,), jnp.float32)
        return carry

    lax.fori_loop(0, NP // 16, zinit, 0)

    ones = jnp.full((16,), 1.0, jnp.float32)

    def i2start(i, p):
        pltpu.async_copy(row_hbm.at[pl.ds(base + i * K, K)], RIDX[p], SEMI[p])
        pltpu.async_copy(col_hbm.at[pl.ds(base + i * K, K)], CIDX[p], SEMI[p])

    def i2wait(p):
        pltpu.make_async_copy(row_hbm.at[pl.ds(0, K)], RIDX[p], SEMI[p]).wait()
        pltpu.make_async_copy(col_hbm.at[pl.ds(0, K)], CIDX[p], SEMI[p]).wait()

    def gstart(i, p):
        pltpu.async_copy(msg_hbm.at[RIDX[p]], ROWS[p], SEMG[p])

    def gwait(b):
        pltpu.make_async_copy(msg_hbm.at[RIDX[b]], ROWS[b], SEMG[b]).wait()

    def sstart(i, b):
        pltpu.async_copy(ROWS[b], agg_sh.at[CIDX[b]], SEMS[b], add=True)

    def swait(p):
        pltpu.make_async_copy(ROWS[p], agg_sh.at[CIDX[p]], SEMS[p]).wait()

    def dodeg(b):
        for j in range(K // 16):
            cc = CIDX[b][pl.ds(j * 16, 16)]
            plsc.addupdate_scatter(deg_v, [cc], ones)

    def full_step(i, b, p, first=False):
        if not first:
            swait(p)
        i2start(i + 1, p)
        gwait(b)
        dodeg(b)
        sstart(i, b)
        i2wait(p)
        gstart(i + 1, p)

    pltpu.sync_copy(row_hbm.at[pl.ds(base, K)], ridx0)
    pltpu.sync_copy(col_hbm.at[pl.ds(base, K)], cidx0)
    gstart(0, 0)
    pltpu.make_async_copy(zeros_hbm.at[pl.ds(0, RPT)],
                          agg_sh.at[pl.ds(s * RPT, RPT)], semg1).wait()
    plsc.subcore_barrier()
    full_step(0, 0, 1, first=True)

    def pair(q, carry):
        i0 = 2 * q + 1
        full_step(i0, 1, 0)
        full_step(i0 + 1, 0, 1)
        return carry

    lax.fori_loop(0, (nc - 3) // 2, pair, 0)

    swait(0)
    i2start(nc - 1, 0)
    gwait(1)
    dodeg(1)
    sstart(nc - 2, 1)
    i2wait(0)
    gstart(nc - 1, 0)
    swait(1)
    gwait(0)
    dodeg(0)
    sstart(nc - 1, 0)
    swait(0)

    pltpu.sync_copy(deg_v, degall_out.at[pl.ds((c * 16 + s) * NP, NP)])
    plsc.subcore_barrier()

    def rstart(t, u):
        pltpu.async_copy(
            degall_out.at[pl.ds((c * 16 + t) * NP + s * RPT, RPT)],
            TMP[u], SEMI[u])

    def rwait(u):
        pltpu.make_async_copy(degall_out.at[pl.ds(0, RPT)], TMP[u],
                              SEMI[u]).wait()

    def radd(u):
        rwait(u)
        for k in range(RPT // 16):
            sl = pl.ds(k * 16, 16)
            acc_v[sl] = acc_v[sl] + TMP[u][sl]

    pltpu.sync_copy(degall_out.at[pl.ds(c * 16 * NP + s * RPT, RPT)], acc_v)
    rstart(1, 1)

    def rpair(q, carry):
        t = 2 * q + 1
        rstart(t + 1, 0)
        radd(1)
        rstart(t + 2, 1)
        radd(0)
        return carry

    lax.fori_loop(0, 7, rpair, 0)
    radd(1)

    pltpu.sync_copy(acc_v, deg_out.at[c, pl.ds(s * RPT, RPT)])

    pltpu.sync_copy(agg_sh.at[pl.ds(s * RPT, RPT)],
                    agg_out.at[c, pl.ds(s * RPT, RPT)])


def _stage2(msg, row, col, zeros):
    kern = functools.partial(
        pl.kernel,
        mesh=plsc.VectorSubcoreMesh(core_axis_name="c", subcore_axis_name="s"),
        out_type=(
            jax.ShapeDtypeStruct((2, NP, D), jnp.float32),
            jax.ShapeDtypeStruct((2, NP), jnp.float32),
            jax.ShapeDtypeStruct((2 * 16 * NP,), jnp.float32),
        ),
        scratch_types=[
            pltpu.VMEM((K,), jnp.int32),
            pltpu.VMEM((K,), jnp.int32),
            pltpu.VMEM((K,), jnp.int32),
            pltpu.VMEM((K,), jnp.int32),
            pltpu.VMEM((K, D), jnp.float32),
            pltpu.VMEM((K, D), jnp.float32),
            pltpu.VMEM((NP,), jnp.float32),
            pltpu.VMEM((RPT,), jnp.float32),
            pltpu.VMEM((RPT,), jnp.float32),
            pltpu.VMEM((RPT,), jnp.float32),
            pltpu.VMEM_SHARED((NP, D), jnp.float32),
        ] + [pltpu.SemaphoreType.DMA] * 6,
        compiler_params=pltpu.CompilerParams(needs_layout_passes=False),
    )(_sc_scatter_body)
    return kern(msg, row, col, zeros)


def _s3_body(x_ref, a0_ref, a1_ref, deg_ref, wf_ref, bf_ref, wo1a_ref,
             wo1b_ref, bo1_ref, wo2_ref, bo2_ref, o_ref):
    s = a0_ref[0] + a1_ref[0]
    aggn = s / jnp.maximum(deg_ref[...], 1.0)
    fx = x_ref[...] + jnp.tanh(aggn @ wf_ref[...] + bf_ref[...])
    o = jnp.maximum(fx @ wo1a_ref[...] + aggn @ wo1b_ref[...] + bo1_ref[...], 0.0)
    o_ref[...] = o @ wo2_ref[...] + bo2_ref[...]


def _stage3(x, agg, degb, Wf, bf, Wo1, bo1, Wo2, bo2):
    grid = N // S3_BLK
    return pl.pallas_call(
        _s3_body,
        grid=(grid,),
        in_specs=[
            pl.BlockSpec((S3_BLK, D), lambda i: (i, 0)),
            pl.BlockSpec((1, S3_BLK, D), lambda i: (0, i, 0)),
            pl.BlockSpec((1, S3_BLK, D), lambda i: (1, i, 0)),
            pl.BlockSpec((S3_BLK, D), lambda i: (i, 0)),
            pl.BlockSpec((D, D), lambda i: (0, 0)),
            pl.BlockSpec((1, D), lambda i: (0, 0)),
            pl.BlockSpec((D, D), lambda i: (0, 0)),
            pl.BlockSpec((D, D), lambda i: (0, 0)),
            pl.BlockSpec((1, D), lambda i: (0, 0)),
            pl.BlockSpec((D, OUT), lambda i: (0, 0)),
            pl.BlockSpec((1, OUT), lambda i: (0, 0)),
        ],
        out_specs=pl.BlockSpec((S3_BLK, OUT), lambda i: (i, 0)),
        out_shape=jax.ShapeDtypeStruct((N, OUT), jnp.float32),
    )(x, agg, agg, degb, Wf, bf.reshape(1, D), Wo1[:D], Wo1[D:],
      bo1.reshape(1, D), Wo2, bo2.reshape(1, OUT))


def kernel(x, edge_index, batch, Wm1, bm1, Wm2, bm2, Wf, bf, Wo1, bo1, Wo2, bo2):
    msg, zeros = _stage1(x, Wm1, bm1, Wm2, bm2)
    agg, deg, _ = _stage2(msg, edge_index[0], edge_index[1], zeros)
    degb = jnp.broadcast_to((deg[0, :N] + deg[1, :N]).reshape(N, 1), (N, D))
    return _stage3(x, agg, degb, Wf, bf, Wo1, bo1, Wo2, bo2)


# speedup vs baseline: 11.2819x; 1.0169x over previous
import functools

import jax
import jax.numpy as jnp
from jax import lax
from jax.experimental import pallas as pl
from jax.experimental.pallas import tpu as pltpu
from jax.experimental.pallas import tpu_sc as plsc

N = 10000
E = 320000
D = 128
OUT = 128

NP = 10240
NW = 32
K = 128
CHUNKS = 79
EPW = CHUNKS * K
LAST_CHUNKS = (E - (NW - 1) * EPW) // K
RPT = NP // 16

S1_BLK = 1000
SZ_BLK = 1024
S3_BLK = 1000


def _s1_body(x_ref, wm1_ref, bm1_ref, wm2_ref, bm2_ref, o_ref, z_ref):
    h = jnp.maximum(x_ref[...] @ wm1_ref[...] + bm1_ref[...], 0.0)
    o_ref[...] = h @ wm2_ref[...] + bm2_ref[...]
    z_ref[...] = jnp.zeros((SZ_BLK, D), jnp.float32)


def _stage1(x, Wm1, bm1, Wm2, bm2):
    grid = N // S1_BLK
    return pl.pallas_call(
        _s1_body,
        grid=(grid,),
        in_specs=[
            pl.BlockSpec((S1_BLK, D), lambda i: (i, 0)),
            pl.BlockSpec((D, D), lambda i: (0, 0)),
            pl.BlockSpec((1, D), lambda i: (0, 0)),
            pl.BlockSpec((D, D), lambda i: (0, 0)),
            pl.BlockSpec((1, D), lambda i: (0, 0)),
        ],
        out_specs=[
            pl.BlockSpec((S1_BLK, D), lambda i: (i, 0)),
            pl.BlockSpec((SZ_BLK, D), lambda i: (i, 0)),
        ],
        out_shape=[
            jax.ShapeDtypeStruct((N, D), jnp.float32),
            jax.ShapeDtypeStruct((NP, D), jnp.float32),
        ],
    )(x, Wm1, bm1.reshape(1, D), Wm2, bm2.reshape(1, D))


def _sc_scatter_body(msg_hbm, row_hbm, col_hbm, zeros_hbm,
                     agg_out, deg_out, degall_out,
                     ridx0, ridx1, cidx0, cidx1, rows0, rows1,
                     deg_v, tmp0_v, tmp1_v, acc_v, agg_sh,
                     semg0, semg1, sems0, sems1, semi0, semi1):
    c = lax.axis_index("c")
    s = lax.axis_index("s")
    wid = s * 2 + c
    base = wid * EPW
    nc = jnp.where(wid == NW - 1, LAST_CHUNKS, CHUNKS)

    RIDX = (ridx0, ridx1)
    CIDX = (cidx0, cidx1)
    ROWS = (rows0, rows1)
    SEMG = (semg0, semg1)
    SEMS = (sems0, sems1)
    SEMI = (semi0, semi1)
    TMP = (tmp0_v, tmp1_v)

    pltpu.async_copy(zeros_hbm.at[pl.ds(s * RPT, RPT)],
                     agg_sh.at[pl.ds(s * RPT, RPT)], semg1)

    def zinit(i, carry):
        deg_v[pl.ds(i * 16, 16)] = jnp.zeros((16,), jnp.float32)
        return carry

    lax.fori_loop(0, NP // 16, zinit, 0)

    ones = jnp.full((16,), 1.0, jnp.float32)

    def i2start(i, p):
        pltpu.async_copy(row_hbm.at[pl.ds(base + i * K, K)], RIDX[p], SEMI[p])
        pltpu.async_copy(col_hbm.at[pl.ds(base + i * K, K)], CIDX[p], SEMI[p])

    def i2wait(p):
        pltpu.make_async_copy(row_hbm.at[pl.ds(0, K)], RIDX[p], SEMI[p]).wait()
        pltpu.make_async_copy(col_hbm.at[pl.ds(0, K)], CIDX[p], SEMI[p]).wait()

    def gstart(i, p):
        pltpu.async_copy(msg_hbm.at[RIDX[p]], ROWS[p], SEMG[p])

    def gwait(b):
        pltpu.make_async_copy(msg_hbm.at[RIDX[b]], ROWS[b], SEMG[b]).wait()

    def sstart(i, b):
        pltpu.async_copy(ROWS[b], agg_sh.at[CIDX[b]], SEMS[b], add=True)

    def swait(p):
        pltpu.make_async_copy(ROWS[p], agg_sh.at[CIDX[p]], SEMS[p]).wait()

    def dodeg(b):
        for j in range(K // 16):
            cc = CIDX[b][pl.ds(j * 16, 16)]
            plsc.addupdate_scatter(deg_v, [cc], ones)

    def full_step(i, b, p, first=False):
        if not first:
            swait(p)
        i2start(i + 1, p)
        gwait(b)
        i2wait(p)
        gstart(i + 1, p)
        dodeg(b)
        sstart(i, b)

    pltpu.sync_copy(row_hbm.at[pl.ds(base, K)], ridx0)
    pltpu.sync_copy(col_hbm.at[pl.ds(base, K)], cidx0)
    gstart(0, 0)
    pltpu.make_async_copy(zeros_hbm.at[pl.ds(0, RPT)],
                          agg_sh.at[pl.ds(s * RPT, RPT)], semg1).wait()
    plsc.subcore_barrier()
    full_step(0, 0, 1, first=True)

    def pair(q, carry):
        i0 = 2 * q + 1
        full_step(i0, 1, 0)
        full_step(i0 + 1, 0, 1)
        return carry

    lax.fori_loop(0, (nc - 3) // 2, pair, 0)

    swait(0)
    i2start(nc - 1, 0)
    gwait(1)
    i2wait(0)
    gstart(nc - 1, 0)
    dodeg(1)
    sstart(nc - 2, 1)
    swait(1)
    gwait(0)
    dodeg(0)
    sstart(nc - 1, 0)
    swait(0)

    pltpu.sync_copy(deg_v, degall_out.at[pl.ds((c * 16 + s) * NP, NP)])
    plsc.subcore_barrier()

    def rstart(t, u):
        pltpu.async_copy(
            degall_out.at[pl.ds((c * 16 + t) * NP + s * RPT, RPT)],
            TMP[u], SEMI[u])

    def rwait(u):
        pltpu.make_async_copy(degall_out.at[pl.ds(0, RPT)], TMP[u],
                              SEMI[u]).wait()

    def radd(u):
        rwait(u)
        for k in range(RPT // 16):
            sl = pl.ds(k * 16, 16)
            acc_v[sl] = acc_v[sl] + TMP[u][sl]

    pltpu.sync_copy(degall_out.at[pl.ds(c * 16 * NP + s * RPT, RPT)], acc_v)
    rstart(1, 1)

    def rpair(q, carry):
        t = 2 * q + 1
        rstart(t + 1, 0)
        radd(1)
        rstart(t + 2, 1)
        radd(0)
        return carry

    lax.fori_loop(0, 7, rpair, 0)
    radd(1)

    pltpu.sync_copy(acc_v, deg_out.at[c, pl.ds(s * RPT, RPT)])

    pltpu.sync_copy(agg_sh.at[pl.ds(s * RPT, RPT)],
                    agg_out.at[c, pl.ds(s * RPT, RPT)])


def _stage2(msg, row, col, zeros):
    kern = functools.partial(
        pl.kernel,
        mesh=plsc.VectorSubcoreMesh(core_axis_name="c", subcore_axis_name="s"),
        out_type=(
            jax.ShapeDtypeStruct((2, NP, D), jnp.float32),
            jax.ShapeDtypeStruct((2, NP), jnp.float32),
            jax.ShapeDtypeStruct((2 * 16 * NP,), jnp.float32),
        ),
        scratch_types=[
            pltpu.VMEM((K,), jnp.int32),
            pltpu.VMEM((K,), jnp.int32),
            pltpu.VMEM((K,), jnp.int32),
            pltpu.VMEM((K,), jnp.int32),
            pltpu.VMEM((K, D), jnp.float32),
            pltpu.VMEM((K, D), jnp.float32),
            pltpu.VMEM((NP,), jnp.float32),
            pltpu.VMEM((RPT,), jnp.float32),
            pltpu.VMEM((RPT,), jnp.float32),
            pltpu.VMEM((RPT,), jnp.float32),
            pltpu.VMEM_SHARED((NP, D), jnp.float32),
        ] + [pltpu.SemaphoreType.DMA] * 6,
        compiler_params=pltpu.CompilerParams(needs_layout_passes=False, disable_bounds_checks=True, disable_semaphore_checks=True),
    )(_sc_scatter_body)
    return kern(msg, row, col, zeros)


def _s3_body(x_ref, a0_ref, a1_ref, deg_ref, wf_ref, bf_ref, wo1a_ref,
             wo1b_ref, bo1_ref, wo2_ref, bo2_ref, o_ref):
    s = a0_ref[0] + a1_ref[0]
    aggn = s / jnp.maximum(deg_ref[...], 1.0)
    fx = x_ref[...] + jnp.tanh(aggn @ wf_ref[...] + bf_ref[...])
    o = jnp.maximum(fx @ wo1a_ref[...] + aggn @ wo1b_ref[...] + bo1_ref[...], 0.0)
    o_ref[...] = o @ wo2_ref[...] + bo2_ref[...]


def _stage3(x, agg, degb, Wf, bf, Wo1, bo1, Wo2, bo2):
    grid = N // S3_BLK
    return pl.pallas_call(
        _s3_body,
        grid=(grid,),
        in_specs=[
            pl.BlockSpec((S3_BLK, D), lambda i: (i, 0)),
            pl.BlockSpec((1, S3_BLK, D), lambda i: (0, i, 0)),
            pl.BlockSpec((1, S3_BLK, D), lambda i: (1, i, 0)),
            pl.BlockSpec((S3_BLK, D), lambda i: (i, 0)),
            pl.BlockSpec((D, D), lambda i: (0, 0)),
            pl.BlockSpec((1, D), lambda i: (0, 0)),
            pl.BlockSpec((D, D), lambda i: (0, 0)),
            pl.BlockSpec((D, D), lambda i: (0, 0)),
            pl.BlockSpec((1, D), lambda i: (0, 0)),
            pl.BlockSpec((D, OUT), lambda i: (0, 0)),
            pl.BlockSpec((1, OUT), lambda i: (0, 0)),
        ],
        out_specs=pl.BlockSpec((S3_BLK, OUT), lambda i: (i, 0)),
        out_shape=jax.ShapeDtypeStruct((N, OUT), jnp.float32),
    )(x, agg, agg, degb, Wf, bf.reshape(1, D), Wo1[:D], Wo1[D:],
      bo1.reshape(1, D), Wo2, bo2.reshape(1, OUT))


def kernel(x, edge_index, batch, Wm1, bm1, Wm2, bm2, Wf, bf, Wo1, bo1, Wo2, bo2):
    msg, zeros = _stage1(x, Wm1, bm1, Wm2, bm2)
    agg, deg, _ = _stage2(msg, edge_index[0], edge_index[1], zeros)
    degb = jnp.broadcast_to((deg[0, :N] + deg[1, :N]).reshape(N, 1), (N, D))
    return _stage3(x, agg, degb, Wf, bf, Wo1, bo1, Wo2, bo2)
